# Initial kernel scaffold; baseline (speedup 1.0000x reference)
import jax
import jax.numpy as jnp
from jax.experimental import pallas as pl


def kernel(cell_features, tissue_features, cluster_labels, tissue_batch, bu_in_w, bu_in_b, bu_out_w, bu_out_b, td_in_w, td_in_b, td_out_w, td_out_b):
    raise NotImplementedError("write your pallas kernel here")


# speedup vs baseline: 19.5329x; 19.5329x over previous
import functools
import numpy as np
import jax
import jax.numpy as jnp
from jax.experimental import pallas as pl
from jax.experimental.pallas import tpu as pltpu

H = 8


def _bu_body(T, D, B, nblk,
             x_ref, lab_ref, tis_ref,
             wqT, bq, wkvT, bkv,
             wboT, bbo, wvtdT, bvtd, wtoT, bto,
             tissue_out, u_out,
             q_s, m_s, d_s, num_s):
    dh = D // H
    i = pl.program_id(0)

    @pl.when(i == 0)
    def _init():
        q_s[...] = jnp.dot(tis_ref[...], wqT[...],
                           preferred_element_type=jnp.float32) + bq[...]
        m_s[...] = jnp.full((1, H), -1e30, jnp.float32)
        d_s[...] = jnp.zeros((T, H), jnp.float32)
        num_s[...] = jnp.zeros((T, D), jnp.float32)

    X = x_ref[...]
    Xb = X.astype(jnp.bfloat16)
    lab = lab_ref[0, 0, :]
    P = (lab[:, None] == jax.lax.broadcasted_iota(jnp.int32, (B, T), 1)
         ).astype(jnp.bfloat16)

    E = (jax.lax.broadcasted_iota(jnp.int32, (D, H), 0) // dh
         == jax.lax.broadcasted_iota(jnp.int32, (D, H), 1)
         ).astype(jnp.bfloat16)
    ET = (jax.lax.broadcasted_iota(jnp.int32, (H, D), 1) // dh
          == jax.lax.broadcasted_iota(jnp.int32, (H, D), 0)
          ).astype(jnp.bfloat16)

    KV = jnp.dot(Xb, wkvT[...], preferred_element_type=jnp.float32) + bkv[...]
    K = KV[:, :D]
    V = KV[:, D:]
    Qg = jnp.dot(P, q_s[...].astype(jnp.bfloat16),
                 preferred_element_type=jnp.float32)
    s = jnp.dot((K * Qg).astype(jnp.bfloat16), E,
                preferred_element_type=jnp.float32)
    s = s * (1.0 / float(np.sqrt(dh)))

    mb = jnp.max(s, axis=0, keepdims=True)
    new_m = jnp.maximum(m_s[...], mb)
    alpha = jnp.exp(m_s[...] - new_m)
    ex = jnp.exp(s - new_m)

    ex_d = jnp.dot(ex.astype(jnp.bfloat16), ET,
                   preferred_element_type=jnp.float32)
    EV = (V * ex_d).astype(jnp.bfloat16)
    G = jnp.concatenate([EV, ex.astype(jnp.bfloat16)], axis=1)
    R = jax.lax.dot_general(P, G, (((0,), (0,)), ((), ())),
                            preferred_element_type=jnp.float32)
    alpha_d = jnp.broadcast_to(alpha[:, :, None], (1, H, dh)).reshape(1, D)
    num_s[...] = num_s[...] * alpha_d + R[:, :D]
    d_s[...] = d_s[...] * alpha + R[:, D:]
    m_s[...] = new_m

    @pl.when(i == nblk - 1)
    def _finish():
        d = d_s[...]
        dd = jnp.maximum(d, 1e-30)
        d_full = jnp.broadcast_to(dd[:, :, None], (T, H, dh)).reshape(T, D)
        att = jnp.dot(num_s[...] / d_full, wboT[...],
                      preferred_element_type=jnp.float32) + bbo[...]
        tn = jnp.where(d[:, :1] > 0, att, tis_ref[...])
        tissue_out[...] = tn
        vt = jnp.dot(tn, wvtdT[...],
                     preferred_element_type=jnp.float32) + bvtd[...]
        u_out[...] = jnp.dot(vt, wtoT[...],
                             preferred_element_type=jnp.float32) + bto[...]


def _td_body(B, T, u_ref, lab_ref, out_ref):
    lab = lab_ref[0, 0, :]
    P = (lab[:, None] == jax.lax.broadcasted_iota(jnp.int32, (B, T), 1)
         ).astype(jnp.bfloat16)
    out_ref[...] = jnp.dot(P, u_ref[...], preferred_element_type=jnp.float32)


def kernel(cell_features, tissue_features, cluster_labels, tissue_batch,
           bu_in_w, bu_in_b, bu_out_w, bu_out_b,
           td_in_w, td_in_b, td_out_w, td_out_b):
    N, D = cell_features.shape
    T = tissue_features.shape[0]
    B = 2000
    assert N % B == 0 and B % 8 == 0
    nblk = N // B

    labels = cluster_labels.astype(jnp.int32).reshape(nblk, 1, B)

    d = D
    wqT = bu_in_w[:d].T
    wkvT = jnp.concatenate([bu_in_w[d:2 * d].T, bu_in_w[2 * d:].T],
                           axis=1).astype(jnp.bfloat16)
    bq = bu_in_b[:d].reshape(1, d)
    bkv = bu_in_b[d:].reshape(1, 2 * d)
    wboT = bu_out_w.T
    bbo = bu_out_b.reshape(1, d)
    wvtdT = td_in_w[2 * d:].T
    bvtd = td_in_b[2 * d:].reshape(1, d)
    wtoT = td_out_w.T
    bto = td_out_b.reshape(1, d)

    full = lambda shape: pl.BlockSpec(shape, lambda i: tuple(0 for _ in shape))

    tissue_new, u = pl.pallas_call(
        functools.partial(_bu_body, T, D, B, nblk),
        grid=(nblk,),
        in_specs=[
            pl.BlockSpec((B, D), lambda i: (i, 0)),
            pl.BlockSpec((1, 1, B), lambda i: (i, 0, 0)),
            full((T, D)),
            full((D, D)), full((1, D)),
            full((D, 2 * D)), full((1, 2 * D)),
            full((D, D)), full((1, D)),
            full((D, D)), full((1, D)),
            full((D, D)), full((1, D)),
        ],
        out_specs=[full((T, D)), full((T, D))],
        out_shape=[
            jax.ShapeDtypeStruct((T, D), jnp.float32),
            jax.ShapeDtypeStruct((T, D), jnp.float32),
        ],
        scratch_shapes=[
            pltpu.VMEM((T, D), jnp.float32),
            pltpu.VMEM((1, H), jnp.float32),
            pltpu.VMEM((T, H), jnp.float32),
            pltpu.VMEM((T, D), jnp.float32),
        ],
    )(cell_features, labels, tissue_features,
      wqT, bq, wkvT, bkv, wboT, bbo, wvtdT, bvtd, wtoT, bto)

    cell_new = pl.pallas_call(
        functools.partial(_td_body, B, T),
        grid=(nblk,),
        in_specs=[
            full((T, D)),
            pl.BlockSpec((1, 1, B), lambda i: (i, 0, 0)),
        ],
        out_specs=pl.BlockSpec((B, D), lambda i: (i, 0)),
        out_shape=jax.ShapeDtypeStruct((N, D), jnp.float32),
    )(u.astype(jnp.bfloat16), labels)

    return cell_new, tissue_new
